# baseline (speedup 1.0000x reference)
import functools

import jax
import jax.numpy as jnp
from jax import lax
from jax.experimental import pallas as pl
from jax.experimental.pallas import tpu as pltpu
from jax.experimental.pallas import tpu_sc as plsc

N_NODES = 10000
D = 128
DH = D // 2
E = 320000
NC = 2
NS = 16
CH = 128
EPT = 20480
NCHUNK = EPT // CH
E_PAD = EPT * NS
N_ACC = 10240
RPT = N_ACC // NS
TRASH = N_NODES
NBUF = 5
KAH = 3

_MESH = plsc.VectorSubcoreMesh(core_axis_name="c", subcore_axis_name="s")


def _fill(ref, val):
    rows, cols = ref.shape
    v = jnp.full((16,), val, jnp.float32)

    def body(i, carry):
        for cblk in range(cols // 16):
            ref[i, pl.ds(cblk * 16, 16)] = v
        return carry

    lax.fori_loop(0, rows, body, 0)


@functools.partial(
    pl.kernel,
    mesh=_MESH,
    out_type=jax.ShapeDtypeStruct((NC, N_ACC, D), jnp.float32),
    scratch_types=[
        pltpu.VMEM((NCHUNK, CH), jnp.int32),
        pltpu.VMEM((CH, D), jnp.float32),
        pltpu.SemaphoreType.DMA,
        pltpu.VMEM_SHARED((N_ACC, D), jnp.float32),
    ],
)
def _deg_kernel(dst_hbm, degp_hbm, dst_v, buf_v, sem, acc_sh):
    c = lax.axis_index("c")
    s = lax.axis_index("s")
    _fill(buf_v, 0.0)
    for k in range(RPT // CH):
        pltpu.sync_copy(buf_v, acc_sh.at[pl.ds(s * RPT + k * CH, CH)])
    plsc.subcore_barrier()
    pltpu.sync_copy(dst_hbm.at[s], dst_v)
    _fill(buf_v, 1.0)
    base = c * (NCHUNK // NC)

    def chunk(j, carry):
        pltpu.make_async_copy(buf_v, acc_sh.at[dst_v.at[base + j]], sem).start(
            add=True
        )
        return carry

    lax.fori_loop(0, NCHUNK // NC, chunk, 0)

    def drain(j, carry):
        pltpu.make_async_copy(buf_v, acc_sh.at[dst_v.at[base + j]], sem).wait()
        return carry

    lax.fori_loop(0, NCHUNK // NC, drain, 0)
    plsc.subcore_barrier()
    pltpu.sync_copy(
        acc_sh.at[pl.ds(s * RPT, RPT)], degp_hbm.at[c, pl.ds(s * RPT, RPT)]
    )


@functools.partial(
    pl.kernel,
    mesh=_MESH,
    out_type=jax.ShapeDtypeStruct((NC, N_ACC, DH), jnp.float32),
    compiler_params=pltpu.CompilerParams(use_tc_tiling_on_sc=False),
    scratch_types=[
        pltpu.VMEM((NCHUNK, CH), jnp.int32),
        pltpu.VMEM((NCHUNK, CH), jnp.int32),
        pltpu.VMEM((CH, DH), jnp.float32),
        pltpu.VMEM((CH, DH), jnp.float32),
        pltpu.VMEM((CH, DH), jnp.float32),
        pltpu.VMEM((CH, DH), jnp.float32),
        pltpu.VMEM((CH, DH), jnp.float32),
        pltpu.SemaphoreType.DMA,
        pltpu.SemaphoreType.DMA,
        pltpu.SemaphoreType.DMA,
        pltpu.SemaphoreType.DMA,
        pltpu.SemaphoreType.DMA,
        pltpu.SemaphoreType.DMA,
        pltpu.SemaphoreType.DMA,
        pltpu.SemaphoreType.DMA,
        pltpu.SemaphoreType.DMA,
        pltpu.SemaphoreType.DMA,
        pltpu.VMEM_SHARED((N_ACC, DH), jnp.float32),
    ],
)
def _scatter_kernel(
    src_hbm, dst_hbm, g_hbm, acc_hbm, src_v, dst_v, r0, r1, r2, r3, r4,
    g0, g1, g2, g3, g4, s0, s1, s2, s3, s4, acc_sh,
):
    rows = (r0, r1, r2, r3, r4)
    gsem = (g0, g1, g2, g3, g4)
    ssem = (s0, s1, s2, s3, s4)
    c = lax.axis_index("c")
    s = lax.axis_index("s")
    _fill(rows[0], 0.0)
    for k in range(RPT // CH):
        pltpu.sync_copy(rows[0], acc_sh.at[pl.ds(s * RPT + k * CH, CH)])
    plsc.subcore_barrier()
    pltpu.sync_copy(src_hbm.at[s], src_v)
    pltpu.sync_copy(dst_hbm.at[s], dst_v)
    g_tab = g_hbm.at[c]

    def gather(j, b):
        pltpu.make_async_copy(g_tab.at[src_v.at[j]], rows[b], gsem[b]).start()

    def gwait(j, b):
        pltpu.make_async_copy(g_tab.at[src_v.at[j]], rows[b], gsem[b]).wait()

    def scat_start(j, b):
        pltpu.make_async_copy(rows[b], acc_sh.at[dst_v.at[j]], ssem[b]).start(add=True)

    def scat_wait(j, b):
        pltpu.make_async_copy(rows[b], acc_sh.at[dst_v.at[j]], ssem[b]).wait()

    SLACK = NBUF - KAH

    def step(j, slot, with_wait=True, with_gather=True):
        if with_wait:
            scat_wait(j - SLACK, (slot + NBUF - SLACK) % NBUF)
        if with_gather:
            gather(j + KAH, (slot + KAH) % NBUF)
        gwait(j, slot)
        scat_start(j, slot)

    for b in range(KAH):
        gather(b, b)
    for j in range(SLACK):
        step(j, j % NBUF, with_wait=False)

    def body(jj, carry):
        for b in range(NBUF):
            step(SLACK + jj * NBUF + b, (SLACK + b) % NBUF)
        return carry

    lax.fori_loop(0, (NCHUNK - NBUF) // NBUF, body, 0)
    for j in range(NCHUNK - KAH, NCHUNK):
        step(j, j % NBUF, with_gather=False)
    for j in range(NCHUNK - SLACK, NCHUNK):
        scat_wait(j, j % NBUF)
    plsc.subcore_barrier()
    pltpu.sync_copy(
        acc_sh.at[pl.ds(s * RPT, RPT)], acc_hbm.at[c, pl.ds(s * RPT, RPT)]
    )


BR = 1000
GRID = N_NODES // BR


def _dinv_of(degp_ref):
    deg = degp_ref[0, :, 0] + degp_ref[1, :, 0] + 1.0
    return lax.rsqrt(deg)


def _split_store(ref, h):
    ref[0] = h[:, :DH]
    ref[1] = h[:, DH:]


def _cat(ref):
    return jnp.concatenate([ref[0], ref[1]], axis=-1)


def _tc_pre_body(x_ref, w_ref, degp_ref, g_ref):
    dinv = _dinv_of(degp_ref)
    h = jnp.dot(x_ref[...], w_ref[...], preferred_element_type=jnp.float32)
    _split_store(g_ref, h * dinv[:, None])


def _tc_mid_body(acc_ref, g1_ref, degp_ref, w_ref, b_ref, g2_ref):
    dinv = _dinv_of(degp_ref)
    t = (_cat(acc_ref) + _cat(g1_ref)) * dinv[:, None] + b_ref[...]
    z = jnp.maximum(t, 0.0)
    h2 = jnp.dot(z, w_ref[...], preferred_element_type=jnp.float32)
    _split_store(g2_ref, h2 * dinv[:, None])


def _tc_post_body(acc_ref, g2_ref, degp_ref, b_ref, out_ref):
    dinv = _dinv_of(degp_ref)
    out_ref[...] = (_cat(acc_ref) + _cat(g2_ref)) * dinv[:, None] + b_ref[...]


_ROWS = pl.BlockSpec((BR, D), lambda i: (i, 0))
_FULLW = pl.BlockSpec((D, D), lambda i: (0, 0))
_DEGP = pl.BlockSpec((NC, BR, D), lambda i: (0, i, 0))
_SPLIT = pl.BlockSpec((NC, BR, DH), lambda i: (0, i, 0))
_BIAS = pl.BlockSpec((1, D), lambda i: (0, 0))
_SPLIT_SHAPE = jax.ShapeDtypeStruct((NC, N_NODES, DH), jnp.float32)

_tc_pre = pl.pallas_call(
    _tc_pre_body,
    grid=(GRID,),
    in_specs=[_ROWS, _FULLW, _DEGP],
    out_specs=_SPLIT,
    out_shape=_SPLIT_SHAPE,
)

_tc_mid = pl.pallas_call(
    _tc_mid_body,
    grid=(GRID,),
    in_specs=[_SPLIT, _SPLIT, _DEGP, _FULLW, _BIAS],
    out_specs=_SPLIT,
    out_shape=_SPLIT_SHAPE,
)

_tc_post = pl.pallas_call(
    _tc_post_body,
    grid=(GRID,),
    in_specs=[_SPLIT, _SPLIT, _DEGP, _BIAS],
    out_specs=_ROWS,
    out_shape=jax.ShapeDtypeStruct((N_NODES, D), jnp.float32),
)


def kernel(x, edge_index, W1, b1, W2, b2):
    src = edge_index[0].astype(jnp.int32)
    dst = edge_index[1].astype(jnp.int32)
    pad = E_PAD - E
    srcp = jnp.concatenate([src, jnp.zeros((pad,), jnp.int32)]).reshape(NS, NCHUNK, CH)
    dstp = jnp.concatenate([dst, jnp.full((pad,), TRASH, jnp.int32)]).reshape(
        NS, NCHUNK, CH
    )
    degp = _deg_kernel(dstp)
    g1 = _tc_pre(x, W1, degp)
    acc1 = _scatter_kernel(srcp, dstp, g1)
    g2 = _tc_mid(acc1, g1, degp, W2, b1.reshape(1, D))
    acc2 = _scatter_kernel(srcp, dstp, g2)
    out = _tc_post(acc2, g2, degp, b2.reshape(1, D))
    return out


# speedup vs baseline: 1.8446x; 1.8446x over previous
import functools

import jax
import jax.numpy as jnp
from jax import lax
from jax.experimental import pallas as pl
from jax.experimental.pallas import tpu as pltpu
from jax.experimental.pallas import tpu_sc as plsc

N_NODES = 10000
D = 128
DH = D // 2
E = 320000
NC = 2
NS = 16
CH = 128
EPT = 20480
NCHUNK = EPT // CH
E_PAD = EPT * NS
N_ACC = 10240
RPT = N_ACC // NS
TRASH = N_NODES
NBUF = 4
KAH = 2

_MESH = plsc.VectorSubcoreMesh(core_axis_name="c", subcore_axis_name="s")


def _fill(ref, val):
    rows, cols = ref.shape
    v = jnp.full((16,), val, jnp.float32)

    def body(i, carry):
        for cblk in range(cols // 16):
            ref[i, pl.ds(cblk * 16, 16)] = v
        return carry

    lax.fori_loop(0, rows, body, 0)


@functools.partial(
    pl.kernel,
    mesh=_MESH,
    out_type=jax.ShapeDtypeStruct((NC, N_ACC, D), jnp.float32),
    scratch_types=[
        pltpu.VMEM((NCHUNK, CH), jnp.int32),
        pltpu.VMEM((CH, D), jnp.float32),
        pltpu.SemaphoreType.DMA,
        pltpu.VMEM_SHARED((N_ACC, D), jnp.float32),
    ],
)
def _deg_kernel(dst_hbm, degp_hbm, dst_v, buf_v, sem, acc_sh):
    c = lax.axis_index("c")
    s = lax.axis_index("s")
    _fill(buf_v, 0.0)
    for k in range(RPT // CH):
        pltpu.sync_copy(buf_v, acc_sh.at[pl.ds(s * RPT + k * CH, CH)])
    plsc.subcore_barrier()
    pltpu.sync_copy(dst_hbm.at[s], dst_v)
    _fill(buf_v, 1.0)
    base = c * (NCHUNK // NC)

    def chunk(j, carry):
        pltpu.make_async_copy(buf_v, acc_sh.at[dst_v.at[base + j]], sem).start(
            add=True
        )
        return carry

    lax.fori_loop(0, NCHUNK // NC, chunk, 0)

    def drain(j, carry):
        pltpu.make_async_copy(buf_v, acc_sh.at[dst_v.at[base + j]], sem).wait()
        return carry

    lax.fori_loop(0, NCHUNK // NC, drain, 0)
    plsc.subcore_barrier()
    pltpu.sync_copy(
        acc_sh.at[pl.ds(s * RPT, RPT)], degp_hbm.at[c, pl.ds(s * RPT, RPT)]
    )


GB = 16
NBLK = NCHUNK // GB
GROWS = N_NODES // NS


@functools.partial(
    pl.kernel,
    mesh=_MESH,
    out_type=jax.ShapeDtypeStruct((NC, N_ACC, DH), jnp.float32),
    compiler_params=pltpu.CompilerParams(use_tc_tiling_on_sc=False),
    scratch_types=[
        pltpu.VMEM((GB, CH), jnp.int32),
        pltpu.VMEM((GB, CH), jnp.int32),
        pltpu.VMEM((GB, CH), jnp.int32),
        pltpu.VMEM((GB, CH), jnp.int32),
        pltpu.VMEM((CH, DH), jnp.float32),
        pltpu.VMEM((CH, DH), jnp.float32),
        pltpu.VMEM((CH, DH), jnp.float32),
        pltpu.VMEM((CH, DH), jnp.float32),
        pltpu.SemaphoreType.DMA,
        pltpu.SemaphoreType.DMA,
        pltpu.SemaphoreType.DMA,
        pltpu.SemaphoreType.DMA,
        pltpu.SemaphoreType.DMA,
        pltpu.SemaphoreType.DMA,
        pltpu.SemaphoreType.DMA,
        pltpu.SemaphoreType.DMA,
        pltpu.SemaphoreType.DMA,
        pltpu.SemaphoreType.DMA,
        pltpu.VMEM_SHARED((N_NODES, DH), jnp.float32),
        pltpu.VMEM_SHARED((N_ACC, DH), jnp.float32),
    ],
)
def _scatter_kernel(
    src_hbm, dst_hbm, g_hbm, acc_hbm, si0, si1, di0, di1, r0, r1, r2, r3,
    g0, g1, g2, g3, s0, s1, s2, s3, i0, i1, gtab_sh, acc_sh,
):
    sidx = (si0, si1)
    didx = (di0, di1)
    rows = (r0, r1, r2, r3)
    gsem = (g0, g1, g2, g3)
    ssem = (s0, s1, s2, s3)
    isem = (i0, i1)
    c = lax.axis_index("c")
    s = lax.axis_index("s")
    _fill(rows[0], 0.0)
    for k in range(RPT // CH):
        pltpu.sync_copy(rows[0], acc_sh.at[pl.ds(s * RPT + k * CH, CH)])
    pltpu.sync_copy(
        g_hbm.at[c, pl.ds(s * GROWS, GROWS)], gtab_sh.at[pl.ds(s * GROWS, GROWS)]
    )
    plsc.subcore_barrier()

    def gather(sref, r, b):
        pltpu.make_async_copy(gtab_sh.at[sref.at[r]], rows[b], gsem[b]).start()

    def gwait(sref, r, b):
        pltpu.make_async_copy(gtab_sh.at[sref.at[r]], rows[b], gsem[b]).wait()

    def scat_start(dref, r, b):
        pltpu.make_async_copy(rows[b], acc_sh.at[dref.at[r]], ssem[b]).start(add=True)

    def scat_wait(dref, r, b):
        pltpu.make_async_copy(rows[b], acc_sh.at[dref.at[r]], ssem[b]).wait()

    def idx_load(blk, par, wait):
        cp1 = pltpu.make_async_copy(src_hbm.at[s * NBLK + blk], sidx[par], isem[par])
        cp2 = pltpu.make_async_copy(dst_hbm.at[s * NBLK + blk], didx[par], isem[par])
        if wait:
            cp1.wait()
            cp2.wait()
        else:
            cp1.start()
            cp2.start()

    SLACK = NBUF - KAH

    def block_body(blk, par, first=False, last=False):
        cs, cd = sidx[par], didx[par]
        ns_, nd = sidx[1 - par], didx[1 - par]
        for i in range(GB):
            slot = i % NBUF
            if not (first and i < SLACK):
                if i < SLACK:
                    scat_wait(nd, i - SLACK + GB, (slot - SLACK) % NBUF)
                else:
                    scat_wait(cd, i - SLACK, (slot - SLACK) % NBUF)
            if i == SLACK and not last:
                idx_load(blk + 1, 1 - par, wait=False)
            if i == GB - KAH and not last:
                idx_load(blk + 1, 1 - par, wait=True)
            if not (last and i >= GB - KAH):
                if i < GB - KAH:
                    gather(cs, i + KAH, (slot + KAH) % NBUF)
                else:
                    gather(ns_, i + KAH - GB, (slot + KAH) % NBUF)
            gwait(cs, i, slot)
            scat_start(cd, i, slot)

    idx_load(0, 0, wait=False)
    idx_load(0, 0, wait=True)
    for b in range(KAH):
        gather(sidx[0], b, b)
    block_body(0, 0, first=True)

    def body(jj, carry):
        block_body(1 + 2 * jj, 1)
        block_body(2 + 2 * jj, 0)
        return carry

    lax.fori_loop(0, (NBLK - 4) // 2, body, 0)
    block_body(NBLK - 3, 1)
    block_body(NBLK - 2, 0)
    block_body(NBLK - 1, 1, last=True)
    for i in range(GB - SLACK, GB):
        scat_wait(didx[1], i, i % NBUF)
    plsc.subcore_barrier()
    pltpu.sync_copy(
        acc_sh.at[pl.ds(s * RPT, RPT)], acc_hbm.at[c, pl.ds(s * RPT, RPT)]
    )


BR = 1000
GRID = N_NODES // BR


def _dinv_of(degp_ref):
    deg = degp_ref[0, :, 0] + degp_ref[1, :, 0] + 1.0
    return lax.rsqrt(deg)


def _split_store(ref, h):
    ref[0] = h[:, :DH]
    ref[1] = h[:, DH:]


def _cat(ref):
    return jnp.concatenate([ref[0], ref[1]], axis=-1)


def _tc_pre_body(x_ref, w_ref, degp_ref, g_ref):
    dinv = _dinv_of(degp_ref)
    h = jnp.dot(x_ref[...], w_ref[...], preferred_element_type=jnp.float32)
    _split_store(g_ref, h * dinv[:, None])


def _tc_mid_body(acc_ref, g1_ref, degp_ref, w_ref, b_ref, g2_ref):
    dinv = _dinv_of(degp_ref)
    t = (_cat(acc_ref) + _cat(g1_ref)) * dinv[:, None] + b_ref[...]
    z = jnp.maximum(t, 0.0)
    h2 = jnp.dot(z, w_ref[...], preferred_element_type=jnp.float32)
    _split_store(g2_ref, h2 * dinv[:, None])


def _tc_post_body(acc_ref, g2_ref, degp_ref, b_ref, out_ref):
    dinv = _dinv_of(degp_ref)
    out_ref[...] = (_cat(acc_ref) + _cat(g2_ref)) * dinv[:, None] + b_ref[...]


_ROWS = pl.BlockSpec((BR, D), lambda i: (i, 0))
_FULLW = pl.BlockSpec((D, D), lambda i: (0, 0))
_DEGP = pl.BlockSpec((NC, BR, D), lambda i: (0, i, 0))
_SPLIT = pl.BlockSpec((NC, BR, DH), lambda i: (0, i, 0))
_BIAS = pl.BlockSpec((1, D), lambda i: (0, 0))
_SPLIT_SHAPE = jax.ShapeDtypeStruct((NC, N_NODES, DH), jnp.float32)

_tc_pre = pl.pallas_call(
    _tc_pre_body,
    grid=(GRID,),
    in_specs=[_ROWS, _FULLW, _DEGP],
    out_specs=_SPLIT,
    out_shape=_SPLIT_SHAPE,
)

_tc_mid = pl.pallas_call(
    _tc_mid_body,
    grid=(GRID,),
    in_specs=[_SPLIT, _SPLIT, _DEGP, _FULLW, _BIAS],
    out_specs=_SPLIT,
    out_shape=_SPLIT_SHAPE,
)

_tc_post = pl.pallas_call(
    _tc_post_body,
    grid=(GRID,),
    in_specs=[_SPLIT, _SPLIT, _DEGP, _BIAS],
    out_specs=_ROWS,
    out_shape=jax.ShapeDtypeStruct((N_NODES, D), jnp.float32),
)


def kernel(x, edge_index, W1, b1, W2, b2):
    src = edge_index[0].astype(jnp.int32)
    dst = edge_index[1].astype(jnp.int32)
    pad = E_PAD - E
    srcp = jnp.concatenate([src, jnp.zeros((pad,), jnp.int32)]).reshape(
        NS * NBLK, GB, CH
    )
    dstflat = jnp.concatenate([dst, jnp.full((pad,), TRASH, jnp.int32)])
    dstp = dstflat.reshape(NS * NBLK, GB, CH)
    degp = _deg_kernel(dstflat.reshape(NS, NCHUNK, CH))
    g1 = _tc_pre(x, W1, degp)
    acc1 = _scatter_kernel(srcp, dstp, g1)
    g2 = _tc_mid(acc1, g1, degp, W2, b1.reshape(1, D))
    acc2 = _scatter_kernel(srcp, dstp, g2)
    out = _tc_post(acc2, g2, degp, b2.reshape(1, D))
    return out


# speedup vs baseline: 2.0069x; 1.0880x over previous
import functools

import jax
import jax.numpy as jnp
from jax import lax
from jax.experimental import pallas as pl
from jax.experimental.pallas import tpu as pltpu
from jax.experimental.pallas import tpu_sc as plsc

N_NODES = 10000
D = 128
DH = D // 2
E = 320000
NC = 2
NS = 16
CH = 128
EPT = 20480
NCHUNK = EPT // CH
E_PAD = EPT * NS
N_ACC = 10240
RPT = N_ACC // NS
TRASH = N_NODES
NBUF = 4
KAH = 2

_MESH = plsc.VectorSubcoreMesh(core_axis_name="c", subcore_axis_name="s")


def _fill(ref, val):
    rows, cols = ref.shape
    v = jnp.full((16,), val, jnp.float32)

    def body(i, carry):
        for cblk in range(cols // 16):
            ref[i, pl.ds(cblk * 16, 16)] = v
        return carry

    lax.fori_loop(0, rows, body, 0)


DW = 32


@functools.partial(
    pl.kernel,
    mesh=_MESH,
    out_type=jax.ShapeDtypeStruct((NC, N_ACC, DW), jnp.float32),
    compiler_params=pltpu.CompilerParams(use_tc_tiling_on_sc=False),
    scratch_types=[
        pltpu.VMEM((NCHUNK, CH), jnp.int32),
        pltpu.VMEM((CH, DW), jnp.float32),
        pltpu.SemaphoreType.DMA,
        pltpu.VMEM_SHARED((N_ACC, DW), jnp.float32),
    ],
)
def _deg_kernel(dst_hbm, degp_hbm, dst_v, buf_v, sem, acc_sh):
    c = lax.axis_index("c")
    s = lax.axis_index("s")
    _fill(buf_v, 0.0)
    for k in range(RPT // CH):
        pltpu.sync_copy(buf_v, acc_sh.at[pl.ds(s * RPT + k * CH, CH)])
    plsc.subcore_barrier()
    pltpu.sync_copy(dst_hbm.at[s], dst_v)
    _fill(buf_v, 1.0)
    base = c * (NCHUNK // NC)

    def chunk(j, carry):
        pltpu.make_async_copy(buf_v, acc_sh.at[dst_v.at[base + j]], sem).start(
            add=True
        )
        return carry

    lax.fori_loop(0, NCHUNK // NC, chunk, 0)

    def drain(j, carry):
        pltpu.make_async_copy(buf_v, acc_sh.at[dst_v.at[base + j]], sem).wait()
        return carry

    lax.fori_loop(0, NCHUNK // NC, drain, 0)
    plsc.subcore_barrier()
    pltpu.sync_copy(
        acc_sh.at[pl.ds(s * RPT, RPT)], degp_hbm.at[c, pl.ds(s * RPT, RPT)]
    )


GB = 16
NBLK = NCHUNK // GB
GROWS = N_NODES // NS


@functools.partial(
    pl.kernel,
    mesh=_MESH,
    out_type=jax.ShapeDtypeStruct((NC, N_ACC, DH), jnp.float32),
    compiler_params=pltpu.CompilerParams(use_tc_tiling_on_sc=False),
    scratch_types=[
        pltpu.VMEM((GB, CH), jnp.int32),
        pltpu.VMEM((GB, CH), jnp.int32),
        pltpu.VMEM((GB, CH), jnp.int32),
        pltpu.VMEM((GB, CH), jnp.int32),
        pltpu.VMEM((CH, DH), jnp.float32),
        pltpu.VMEM((CH, DH), jnp.float32),
        pltpu.VMEM((CH, DH), jnp.float32),
        pltpu.VMEM((CH, DH), jnp.float32),
        pltpu.SemaphoreType.DMA,
        pltpu.SemaphoreType.DMA,
        pltpu.SemaphoreType.DMA,
        pltpu.SemaphoreType.DMA,
        pltpu.SemaphoreType.DMA,
        pltpu.SemaphoreType.DMA,
        pltpu.SemaphoreType.DMA,
        pltpu.SemaphoreType.DMA,
        pltpu.SemaphoreType.DMA,
        pltpu.SemaphoreType.DMA,
        pltpu.VMEM_SHARED((N_NODES, DH), jnp.float32),
        pltpu.VMEM_SHARED((N_ACC, DH), jnp.float32),
    ],
)
def _scatter_kernel(
    src_hbm, dst_hbm, g_hbm, acc_hbm, si0, si1, di0, di1, r0, r1, r2, r3,
    g0, g1, g2, g3, s0, s1, s2, s3, i0, i1, gtab_sh, acc_sh,
):
    sidx = (si0, si1)
    didx = (di0, di1)
    rows = (r0, r1, r2, r3)
    gsem = (g0, g1, g2, g3)
    ssem = (s0, s1, s2, s3)
    isem = (i0, i1)
    c = lax.axis_index("c")
    s = lax.axis_index("s")
    _fill(rows[0], 0.0)
    for k in range(RPT // CH):
        pltpu.sync_copy(rows[0], acc_sh.at[pl.ds(s * RPT + k * CH, CH)])
    pltpu.sync_copy(
        g_hbm.at[c, pl.ds(s * GROWS, GROWS)], gtab_sh.at[pl.ds(s * GROWS, GROWS)]
    )
    plsc.subcore_barrier()

    def gather(sref, r, b):
        pltpu.make_async_copy(gtab_sh.at[sref.at[r]], rows[b], gsem[b]).start()

    def gwait(sref, r, b):
        pltpu.make_async_copy(gtab_sh.at[sref.at[r]], rows[b], gsem[b]).wait()

    def scat_start(dref, r, b):
        pltpu.make_async_copy(rows[b], acc_sh.at[dref.at[r]], ssem[b]).start(add=True)

    def scat_wait(dref, r, b):
        pltpu.make_async_copy(rows[b], acc_sh.at[dref.at[r]], ssem[b]).wait()

    def idx_load(blk, par, wait):
        cp1 = pltpu.make_async_copy(src_hbm.at[s * NBLK + blk], sidx[par], isem[par])
        cp2 = pltpu.make_async_copy(dst_hbm.at[s * NBLK + blk], didx[par], isem[par])
        if wait:
            cp1.wait()
            cp2.wait()
        else:
            cp1.start()
            cp2.start()

    SLACK = NBUF - KAH

    def block_body(blk, par, first=False, last=False):
        cs, cd = sidx[par], didx[par]
        ns_, nd = sidx[1 - par], didx[1 - par]
        for i in range(GB):
            slot = i % NBUF
            if not (first and i < SLACK):
                if i < SLACK:
                    scat_wait(nd, i - SLACK + GB, (slot - SLACK) % NBUF)
                else:
                    scat_wait(cd, i - SLACK, (slot - SLACK) % NBUF)
            if i == SLACK and not last:
                idx_load(blk + 1, 1 - par, wait=False)
            if i == GB - KAH and not last:
                idx_load(blk + 1, 1 - par, wait=True)
            if not (last and i >= GB - KAH):
                if i < GB - KAH:
                    gather(cs, i + KAH, (slot + KAH) % NBUF)
                else:
                    gather(ns_, i + KAH - GB, (slot + KAH) % NBUF)
            gwait(cs, i, slot)
            scat_start(cd, i, slot)

    idx_load(0, 0, wait=False)
    idx_load(0, 0, wait=True)
    for b in range(KAH):
        gather(sidx[0], b, b)
    block_body(0, 0, first=True)

    def body(jj, carry):
        block_body(1 + 2 * jj, 1)
        block_body(2 + 2 * jj, 0)
        return carry

    lax.fori_loop(0, (NBLK - 4) // 2, body, 0)
    block_body(NBLK - 3, 1)
    block_body(NBLK - 2, 0)
    block_body(NBLK - 1, 1, last=True)
    for i in range(GB - SLACK, GB):
        scat_wait(didx[1], i, i % NBUF)
    plsc.subcore_barrier()
    pltpu.sync_copy(
        acc_sh.at[pl.ds(s * RPT, RPT)], acc_hbm.at[c, pl.ds(s * RPT, RPT)]
    )


BR = 1000
GRID = N_NODES // BR


def _dinv_of(degp_ref):
    deg = degp_ref[0, :, 0] + degp_ref[1, :, 0] + 1.0
    return lax.rsqrt(deg)


def _split_store(ref, h):
    ref[0] = h[:, :DH]
    ref[1] = h[:, DH:]


def _cat(ref):
    return jnp.concatenate([ref[0], ref[1]], axis=-1)


def _tc_pre_body(x_ref, w_ref, degp_ref, g_ref):
    dinv = _dinv_of(degp_ref)
    h = jnp.dot(x_ref[...], w_ref[...], preferred_element_type=jnp.float32)
    _split_store(g_ref, h * dinv[:, None])


def _tc_mid_body(acc_ref, g1_ref, degp_ref, w_ref, b_ref, g2_ref):
    dinv = _dinv_of(degp_ref)
    t = (_cat(acc_ref) + _cat(g1_ref)) * dinv[:, None] + b_ref[...]
    z = jnp.maximum(t, 0.0)
    h2 = jnp.dot(z, w_ref[...], preferred_element_type=jnp.float32)
    _split_store(g2_ref, h2 * dinv[:, None])


def _tc_post_body(acc_ref, g2_ref, degp_ref, b_ref, out_ref):
    dinv = _dinv_of(degp_ref)
    out_ref[...] = (_cat(acc_ref) + _cat(g2_ref)) * dinv[:, None] + b_ref[...]


_ROWS = pl.BlockSpec((BR, D), lambda i: (i, 0))
_FULLW = pl.BlockSpec((D, D), lambda i: (0, 0))
_DEGP = pl.BlockSpec((NC, BR, DW), lambda i: (0, i, 0))
_SPLIT = pl.BlockSpec((NC, BR, DH), lambda i: (0, i, 0))
_BIAS = pl.BlockSpec((1, D), lambda i: (0, 0))
_SPLIT_SHAPE = jax.ShapeDtypeStruct((NC, N_NODES, DH), jnp.float32)

_tc_pre = pl.pallas_call(
    _tc_pre_body,
    grid=(GRID,),
    in_specs=[_ROWS, _FULLW, _DEGP],
    out_specs=_SPLIT,
    out_shape=_SPLIT_SHAPE,
)

_tc_mid = pl.pallas_call(
    _tc_mid_body,
    grid=(GRID,),
    in_specs=[_SPLIT, _SPLIT, _DEGP, _FULLW, _BIAS],
    out_specs=_SPLIT,
    out_shape=_SPLIT_SHAPE,
)

_tc_post = pl.pallas_call(
    _tc_post_body,
    grid=(GRID,),
    in_specs=[_SPLIT, _SPLIT, _DEGP, _BIAS],
    out_specs=_ROWS,
    out_shape=jax.ShapeDtypeStruct((N_NODES, D), jnp.float32),
)


def kernel(x, edge_index, W1, b1, W2, b2):
    src = edge_index[0].astype(jnp.int32)
    dst = edge_index[1].astype(jnp.int32)
    pad = E_PAD - E
    srcp = jnp.concatenate([src, jnp.zeros((pad,), jnp.int32)]).reshape(
        NS * NBLK, GB, CH
    )
    dstflat = jnp.concatenate([dst, jnp.full((pad,), TRASH, jnp.int32)])
    dstp = dstflat.reshape(NS * NBLK, GB, CH)
    degp = _deg_kernel(dstflat.reshape(NS, NCHUNK, CH))
    g1 = _tc_pre(x, W1, degp)
    acc1 = _scatter_kernel(srcp, dstp, g1)
    g2 = _tc_mid(acc1, g1, degp, W2, b1.reshape(1, D))
    acc2 = _scatter_kernel(srcp, dstp, g2)
    out = _tc_post(acc2, g2, degp, b2.reshape(1, D))
    return out


# speedup vs baseline: 2.0456x; 1.0193x over previous
import functools

import jax
import jax.numpy as jnp
from jax import lax
from jax.experimental import pallas as pl
from jax.experimental.pallas import tpu as pltpu
from jax.experimental.pallas import tpu_sc as plsc

N_NODES = 10000
D = 128
DH = D // 2
E = 320000
NC = 2
NS = 16
CH = 128
EPT = 20480
NCHUNK = EPT // CH
E_PAD = EPT * NS
N_ACC = 10240
RPT = N_ACC // NS
TRASH = N_NODES
NBUF = 4
KAH = 2

_MESH = plsc.VectorSubcoreMesh(core_axis_name="c", subcore_axis_name="s")


def _fill(ref, val):
    rows, cols = ref.shape
    v = jnp.full((16,), val, jnp.float32)

    def body(i, carry):
        for cblk in range(cols // 16):
            ref[i, pl.ds(cblk * 16, 16)] = v
        return carry

    lax.fori_loop(0, rows, body, 0)


DW = 16


@functools.partial(
    pl.kernel,
    mesh=_MESH,
    out_type=jax.ShapeDtypeStruct((NC, N_ACC, DW), jnp.float32),
    compiler_params=pltpu.CompilerParams(use_tc_tiling_on_sc=False),
    scratch_types=[
        pltpu.VMEM((NCHUNK, CH), jnp.int32),
        pltpu.VMEM((CH, DW), jnp.float32),
        pltpu.SemaphoreType.DMA,
        pltpu.VMEM_SHARED((N_ACC, DW), jnp.float32),
    ],
)
def _deg_kernel(dst_hbm, degp_hbm, dst_v, buf_v, sem, acc_sh):
    c = lax.axis_index("c")
    s = lax.axis_index("s")
    _fill(buf_v, 0.0)
    for k in range(RPT // CH):
        pltpu.sync_copy(buf_v, acc_sh.at[pl.ds(s * RPT + k * CH, CH)])
    plsc.subcore_barrier()
    pltpu.sync_copy(dst_hbm.at[s], dst_v)
    _fill(buf_v, 1.0)
    base = c * (NCHUNK // NC)

    def chunk(j, carry):
        pltpu.make_async_copy(buf_v, acc_sh.at[dst_v.at[base + j]], sem).start(
            add=True
        )
        return carry

    lax.fori_loop(0, NCHUNK // NC, chunk, 0)

    def drain(j, carry):
        pltpu.make_async_copy(buf_v, acc_sh.at[dst_v.at[base + j]], sem).wait()
        return carry

    lax.fori_loop(0, NCHUNK // NC, drain, 0)
    plsc.subcore_barrier()
    pltpu.sync_copy(
        acc_sh.at[pl.ds(s * RPT, RPT)], degp_hbm.at[c, pl.ds(s * RPT, RPT)]
    )


GB = 16
NBLK = NCHUNK // GB
GROWS = N_NODES // NS


@functools.partial(
    pl.kernel,
    mesh=_MESH,
    out_type=jax.ShapeDtypeStruct((NC, N_ACC, DH), jnp.float32),
    compiler_params=pltpu.CompilerParams(use_tc_tiling_on_sc=False),
    scratch_types=[
        pltpu.VMEM((GB, CH), jnp.int32),
        pltpu.VMEM((GB, CH), jnp.int32),
        pltpu.VMEM((GB, CH), jnp.int32),
        pltpu.VMEM((GB, CH), jnp.int32),
        pltpu.VMEM((CH, DH), jnp.float32),
        pltpu.VMEM((CH, DH), jnp.float32),
        pltpu.VMEM((CH, DH), jnp.float32),
        pltpu.VMEM((CH, DH), jnp.float32),
        pltpu.SemaphoreType.DMA,
        pltpu.SemaphoreType.DMA,
        pltpu.SemaphoreType.DMA,
        pltpu.SemaphoreType.DMA,
        pltpu.SemaphoreType.DMA,
        pltpu.SemaphoreType.DMA,
        pltpu.SemaphoreType.DMA,
        pltpu.SemaphoreType.DMA,
        pltpu.SemaphoreType.DMA,
        pltpu.SemaphoreType.DMA,
        pltpu.VMEM_SHARED((N_NODES, DH), jnp.float32),
        pltpu.VMEM_SHARED((N_ACC, DH), jnp.float32),
    ],
)
def _scatter_kernel(
    src_hbm, dst_hbm, g_hbm, acc_hbm, si0, si1, di0, di1, r0, r1, r2, r3,
    g0, g1, g2, g3, s0, s1, s2, s3, i0, i1, gtab_sh, acc_sh,
):
    sidx = (si0, si1)
    didx = (di0, di1)
    rows = (r0, r1, r2, r3)
    gsem = (g0, g1, g2, g3)
    ssem = (s0, s1, s2, s3)
    isem = (i0, i1)
    c = lax.axis_index("c")
    s = lax.axis_index("s")
    _fill(rows[0], 0.0)
    for k in range(RPT // CH):
        pltpu.sync_copy(rows[0], acc_sh.at[pl.ds(s * RPT + k * CH, CH)])
    pltpu.sync_copy(
        g_hbm.at[c, pl.ds(s * GROWS, GROWS)], gtab_sh.at[pl.ds(s * GROWS, GROWS)]
    )
    plsc.subcore_barrier()

    def gather(sref, r, b):
        pltpu.make_async_copy(gtab_sh.at[sref.at[r]], rows[b], gsem[b]).start()

    def gwait(sref, r, b):
        pltpu.make_async_copy(gtab_sh.at[sref.at[r]], rows[b], gsem[b]).wait()

    def scat_start(dref, r, b):
        pltpu.make_async_copy(rows[b], acc_sh.at[dref.at[r]], ssem[b]).start(add=True)

    def scat_wait(dref, r, b):
        pltpu.make_async_copy(rows[b], acc_sh.at[dref.at[r]], ssem[b]).wait()

    def idx_load(blk, par, wait):
        cp1 = pltpu.make_async_copy(src_hbm.at[s * NBLK + blk], sidx[par], isem[par])
        cp2 = pltpu.make_async_copy(dst_hbm.at[s * NBLK + blk], didx[par], isem[par])
        if wait:
            cp1.wait()
            cp2.wait()
        else:
            cp1.start()
            cp2.start()

    SLACK = NBUF - KAH

    def block_body(blk, par, first=False, last=False):
        cs, cd = sidx[par], didx[par]
        ns_, nd = sidx[1 - par], didx[1 - par]
        for i in range(GB):
            slot = i % NBUF
            if not (first and i < SLACK):
                if i < SLACK:
                    scat_wait(nd, i - SLACK + GB, (slot - SLACK) % NBUF)
                else:
                    scat_wait(cd, i - SLACK, (slot - SLACK) % NBUF)
            if i == SLACK and not last:
                idx_load(blk + 1, 1 - par, wait=False)
            if i == GB - KAH and not last:
                idx_load(blk + 1, 1 - par, wait=True)
            if not (last and i >= GB - KAH):
                if i < GB - KAH:
                    gather(cs, i + KAH, (slot + KAH) % NBUF)
                else:
                    gather(ns_, i + KAH - GB, (slot + KAH) % NBUF)
            gwait(cs, i, slot)
            scat_start(cd, i, slot)

    idx_load(0, 0, wait=False)
    idx_load(0, 0, wait=True)
    for b in range(KAH):
        gather(sidx[0], b, b)
    block_body(0, 0, first=True)

    def body(jj, carry):
        block_body(1 + 2 * jj, 1)
        block_body(2 + 2 * jj, 0)
        return carry

    lax.fori_loop(0, (NBLK - 4) // 2, body, 0)
    block_body(NBLK - 3, 1)
    block_body(NBLK - 2, 0)
    block_body(NBLK - 1, 1, last=True)
    for i in range(GB - SLACK, GB):
        scat_wait(didx[1], i, i % NBUF)
    plsc.subcore_barrier()
    pltpu.sync_copy(
        acc_sh.at[pl.ds(s * RPT, RPT)], acc_hbm.at[c, pl.ds(s * RPT, RPT)]
    )


BR = 1000
GRID = N_NODES // BR


def _dinv_of(degp_ref):
    deg = degp_ref[0, :, 0] + degp_ref[1, :, 0] + 1.0
    return lax.rsqrt(deg)


def _split_store(ref, h):
    ref[0] = h[:, :DH]
    ref[1] = h[:, DH:]


def _cat(ref):
    return jnp.concatenate([ref[0], ref[1]], axis=-1)


def _tc_pre_body(x_ref, w_ref, degp_ref, g_ref):
    dinv = _dinv_of(degp_ref)
    h = jnp.dot(x_ref[...], w_ref[...], preferred_element_type=jnp.float32)
    _split_store(g_ref, h * dinv[:, None])


def _tc_mid_body(acc_ref, g1_ref, degp_ref, w_ref, b_ref, g2_ref):
    dinv = _dinv_of(degp_ref)
    t = (_cat(acc_ref) + _cat(g1_ref)) * dinv[:, None] + b_ref[...]
    z = jnp.maximum(t, 0.0)
    h2 = jnp.dot(z, w_ref[...], preferred_element_type=jnp.float32)
    _split_store(g2_ref, h2 * dinv[:, None])


def _tc_post_body(acc_ref, g2_ref, degp_ref, b_ref, out_ref):
    dinv = _dinv_of(degp_ref)
    out_ref[...] = (_cat(acc_ref) + _cat(g2_ref)) * dinv[:, None] + b_ref[...]


_ROWS = pl.BlockSpec((BR, D), lambda i: (i, 0))
_FULLW = pl.BlockSpec((D, D), lambda i: (0, 0))
_DEGP = pl.BlockSpec((NC, BR, DW), lambda i: (0, i, 0))
_SPLIT = pl.BlockSpec((NC, BR, DH), lambda i: (0, i, 0))
_BIAS = pl.BlockSpec((1, D), lambda i: (0, 0))
_SPLIT_SHAPE = jax.ShapeDtypeStruct((NC, N_NODES, DH), jnp.float32)

_tc_pre = pl.pallas_call(
    _tc_pre_body,
    grid=(GRID,),
    in_specs=[_ROWS, _FULLW, _DEGP],
    out_specs=_SPLIT,
    out_shape=_SPLIT_SHAPE,
)

_tc_mid = pl.pallas_call(
    _tc_mid_body,
    grid=(GRID,),
    in_specs=[_SPLIT, _SPLIT, _DEGP, _FULLW, _BIAS],
    out_specs=_SPLIT,
    out_shape=_SPLIT_SHAPE,
)

_tc_post = pl.pallas_call(
    _tc_post_body,
    grid=(GRID,),
    in_specs=[_SPLIT, _SPLIT, _DEGP, _BIAS],
    out_specs=_ROWS,
    out_shape=jax.ShapeDtypeStruct((N_NODES, D), jnp.float32),
)


def kernel(x, edge_index, W1, b1, W2, b2):
    src = edge_index[0].astype(jnp.int32)
    dst = edge_index[1].astype(jnp.int32)
    pad = E_PAD - E
    srcp = jnp.concatenate([src, jnp.zeros((pad,), jnp.int32)]).reshape(
        NS * NBLK, GB, CH
    )
    dstflat = jnp.concatenate([dst, jnp.full((pad,), TRASH, jnp.int32)])
    dstp = dstflat.reshape(NS * NBLK, GB, CH)
    degp = _deg_kernel(dstflat.reshape(NS, NCHUNK, CH))
    g1 = _tc_pre(x, W1, degp)
    acc1 = _scatter_kernel(srcp, dstp, g1)
    g2 = _tc_mid(acc1, g1, degp, W2, b1.reshape(1, D))
    acc2 = _scatter_kernel(srcp, dstp, g2)
    out = _tc_post(acc2, g2, degp, b2.reshape(1, D))
    return out
